# Initial kernel scaffold; baseline (speedup 1.0000x reference)
import jax
import jax.numpy as jnp
from jax.experimental import pallas as pl


def kernel(bucket_matrix, W):
    raise NotImplementedError("write your pallas kernel here")


# speedup vs baseline: 10.8461x; 10.8461x over previous
import functools

import jax
import jax.numpy as jnp
from jax import lax
from jax.experimental import pallas as pl
from jax.experimental.pallas import tpu as pltpu
from jax.experimental.pallas import tpu_sc as plsc

_LANES = 16


def _bias_kernel(B, H, M, NB):
    NW = 32
    total = B * M
    per_w = total // NW
    CH = 2048
    n_chunks = per_w // CH
    per_b_workers = NW // B

    mesh = plsc.VectorSubcoreMesh(core_axis_name="c", subcore_axis_name="s")

    @functools.partial(
        pl.kernel,
        out_type=jax.ShapeDtypeStruct((B, H, M), jnp.float32),
        mesh=mesh,
        compiler_params=pltpu.CompilerParams(needs_layout_passes=False),
        scratch_types=[
            pltpu.VMEM((NB * H,), jnp.float32),
            pltpu.VMEM((CH,), jnp.int32),
            pltpu.VMEM((H, CH), jnp.float32),
            pltpu.SemaphoreType.DMA,
        ],
    )
    def run(bkt_hbm, w_hbm, out_hbm, w_v, idx_v, out_v, sem):
        wid = lax.axis_index("s") * 2 + lax.axis_index("c")
        b = wid // per_b_workers
        m_base = (wid % per_b_workers) * per_w

        pltpu.sync_copy(w_hbm, w_v)

        def chunk_body(c, carry):
            m0 = m_base + c * CH
            pltpu.sync_copy(bkt_hbm.at[b, pl.ds(m0, CH)], idx_v)

            def gather_body(i, carry2):
                idx16 = idx_v[pl.ds(i * _LANES, _LANES)] << 4
                for h in range(H):
                    out_v[h, pl.ds(i * _LANES, _LANES)] = plsc.load_gather(
                        w_v, [idx16 + h]
                    )
                return carry2

            lax.fori_loop(0, CH // _LANES, gather_body, 0, unroll=2)
            pltpu.sync_copy(out_v, out_hbm.at[b, :, pl.ds(m0, CH)])
            return carry

        lax.fori_loop(0, n_chunks, chunk_body, 0)

    return run


def kernel(bucket_matrix, W):
    B, Q, K = bucket_matrix.shape
    NB, H = W.shape
    M = Q * K
    run = _bias_kernel(B, H, M, NB)
    out = run(bucket_matrix.reshape(B, M), W.reshape(-1))
    return out.reshape(B, H, Q, K)
